# Initial kernel scaffold; baseline (speedup 1.0000x reference)
import jax
import jax.numpy as jnp
from jax.experimental import pallas as pl


def kernel(node_feats, edge_feats, edge_index, Wq, bq, Wk, bk, Wv, bv, We, be):
    raise NotImplementedError("write your pallas kernel here")


# speedup vs baseline: 11.8417x; 11.8417x over previous
import functools

import jax
import jax.numpy as jnp
from jax import lax
from jax.experimental import pallas as pl
from jax.experimental.pallas import tpu as pltpu
from jax.experimental.pallas import tpu_sc as plsc

N_NODES = 10000
N_EDGES = 320000
D_IN = 128
D_HEAD = 16
N_HEADS = 8
D_QK = D_HEAD * N_HEADS

NC = 2
NS = 16
NW = NC * NS
EPW = N_EDGES // NW
C = 40
NCHUNK = EPW // C
NP = 10240
RPT = NP // NS


def _proj_body(x_ref, wq_ref, bq_ref, wk_ref, bk_ref, wv_ref, bv_ref,
               q_ref, kv_ref):
    x = x_ref[...]
    q = jnp.dot(x, wq_ref[...], preferred_element_type=jnp.float32) + bq_ref[...]
    q_ref[...] = q * 0.25
    kv_ref[:, :D_QK] = (
        jnp.dot(x, wk_ref[...], preferred_element_type=jnp.float32) + bk_ref[...])
    kv_ref[:, D_QK:] = (
        jnp.dot(x, wv_ref[...], preferred_element_type=jnp.float32) + bv_ref[...])


def _node_proj(node_feats, Wq, bq, Wk, bk, Wv, bv):
    BM = 1000
    grid = (N_NODES // BM,)
    wspec = pl.BlockSpec((D_IN, D_QK), lambda i: (0, 0))
    bspec = pl.BlockSpec((1, D_QK), lambda i: (0, 0))
    return pl.pallas_call(
        _proj_body,
        grid=grid,
        in_specs=[pl.BlockSpec((BM, D_IN), lambda i: (i, 0)),
                  wspec, bspec, wspec, bspec, wspec, bspec],
        out_specs=[pl.BlockSpec((BM, D_QK), lambda i: (i, 0)),
                   pl.BlockSpec((BM, 2 * D_QK), lambda i: (i, 0))],
        out_shape=[jax.ShapeDtypeStruct((N_NODES, D_QK), jnp.float32),
                   jax.ShapeDtypeStruct((N_NODES, 2 * D_QK), jnp.float32)],
    )(node_feats, Wq, bq.reshape(1, -1), Wk, bk.reshape(1, -1),
      Wv, bv.reshape(1, -1))


def _edge_proj_body(x_ref, we_ref, be_ref, pe_ref):
    pe_ref[...] = (
        jnp.dot(x_ref[...], we_ref[...], preferred_element_type=jnp.float32)
        + be_ref[...])


def _edge_proj(edge_feats, We, be):
    BM = 2000
    grid = (N_EDGES // BM,)
    return pl.pallas_call(
        _edge_proj_body,
        grid=grid,
        in_specs=[pl.BlockSpec((BM, D_IN), lambda i: (i, 0)),
                  pl.BlockSpec((D_IN, D_QK), lambda i: (0, 0)),
                  pl.BlockSpec((1, D_QK), lambda i: (0, 0))],
        out_specs=pl.BlockSpec((BM, D_QK), lambda i: (i, 0)),
        out_shape=jax.ShapeDtypeStruct((N_EDGES, D_QK), jnp.float32),
    )(edge_feats, We, be.reshape(1, -1))


def _sc_body(kv_hbm, q_hbm, pe_hbm, src_hbm, dst_hbm,
             eout_hbm, wvp_hbm, zp_hbm,
             src_v, dst_v, kv_v, q_v, pe_v, eout_v, wv_v, z_v, dst8_v,
             wv_acc, z_acc, sem):
    cid = lax.axis_index("c")
    sid = lax.axis_index("s")
    wid = cid * NS + sid
    lane = lax.broadcasted_iota(jnp.int32, (D_HEAD,), 0)

    def zfill(i, _):
        for j in range(D_QK // D_HEAD):
            wv_v[i, pl.ds(j * D_HEAD, D_HEAD)] = jnp.zeros((D_HEAD,),
                                                           jnp.float32)
            z_v[i, pl.ds(j * D_HEAD, D_HEAD)] = jnp.zeros((D_HEAD,),
                                                          jnp.float32)
        return 0
    lax.fori_loop(0, C, zfill, 0)
    for b in range(RPT // C):
        off = sid * RPT + b * C
        pltpu.sync_copy(wv_v, wv_acc.at[pl.ds(off, C)])
    for b in range(RPT // (8 * C)):
        off = sid * (RPT // 8) + b * C
        pltpu.sync_copy(z_v, z_acc.at[pl.ds(off, C)])
    plsc.subcore_barrier()

    def chunk(j, _):
        base = wid * EPW + j * C
        pltpu.sync_copy(src_hbm.at[pl.ds(base, C)], src_v)
        pltpu.sync_copy(dst_hbm.at[pl.ds(base, C)], dst_v)
        pltpu.async_copy(kv_hbm.at[src_v], kv_v, sem).wait()
        pltpu.async_copy(q_hbm.at[dst_v], q_v, sem).wait()
        pltpu.sync_copy(pe_hbm.at[pl.ds(base, C)], pe_v)
        for o in (0, 16, 24):
            dst8_v[pl.ds(o, 16)] = lax.shift_right_logical(
                dst_v[pl.ds(o, 16)], 3)

        def edge(e, _):
            svals = jnp.zeros((D_HEAD,), jnp.float32)
            for h in range(N_HEADS):
                sl = pl.ds(h * D_HEAD, D_HEAD)
                k = kv_v[e, sl]
                v = kv_v[e, pl.ds(D_QK + h * D_HEAD, D_HEAD)]
                q = q_v[e, sl]
                pe = pe_v[e, sl]
                sc = jnp.clip(k * q, -5.0, 5.0) * pe
                eout_v[e, sl] = sc
                tot = sc
                for sh in (8, 4, 2, 1):
                    tot = tot + tot[lane ^ sh]
                svec = jnp.exp(jnp.clip(tot, -5.0, 5.0))
                wv_v[e, sl] = v * svec
                svals = jnp.where(lane == h, svec, svals)
            wbase = (e // 16) * 16
            w = dst_v[pl.ds(wbase, 16)]
            gvecf = (w[jnp.full((16,), e % 16, jnp.int32)] & 7
                     ).astype(jnp.float32)
            for g in range(8):
                ind = jnp.maximum(1.0 - jnp.abs(gvecf - float(g)), 0.0)
                z_v[e, pl.ds(g * D_HEAD, D_HEAD)] = svals * ind
            return 0
        lax.fori_loop(0, C, edge, 0)

        pltpu.sync_copy(eout_v, eout_hbm.at[pl.ds(base, C)])
        pltpu.sync_copy(wv_v, wv_acc.at[dst_v], add=True)
        pltpu.sync_copy(z_v, z_acc.at[dst8_v], add=True)
        return 0
    lax.fori_loop(0, NCHUNK, chunk, 0)

    plsc.subcore_barrier()
    for b in range(RPT // (4 * C)):
        off = sid * RPT + b * 4 * C
        pltpu.sync_copy(wv_acc.at[pl.ds(off, 4 * C)],
                        wvp_hbm.at[pl.ds(cid * NP + off, 4 * C)])
    off = sid * (RPT // 8)
    pltpu.sync_copy(z_acc.at[pl.ds(off, 2 * C)],
                    zp_hbm.at[pl.ds(cid * (NP // 8) + off, 2 * C)])


_sc_edge = functools.partial(
    pl.kernel,
    out_type=[jax.ShapeDtypeStruct((N_EDGES, D_QK), jnp.float32),
              jax.ShapeDtypeStruct((NC * NP, D_QK), jnp.float32),
              jax.ShapeDtypeStruct((NC * (NP // 8), D_QK), jnp.float32)],
    mesh=plsc.VectorSubcoreMesh(core_axis_name="c", subcore_axis_name="s",
                                num_cores=NC, num_subcores=NS),
    scratch_types=[
        pltpu.VMEM((C,), jnp.int32),
        pltpu.VMEM((C,), jnp.int32),
        pltpu.VMEM((C, 2 * D_QK), jnp.float32),
        pltpu.VMEM((C, D_QK), jnp.float32),
        pltpu.VMEM((C, D_QK), jnp.float32),
        pltpu.VMEM((C, D_QK), jnp.float32),
        pltpu.VMEM((C, D_QK), jnp.float32),
        pltpu.VMEM((C, D_QK), jnp.float32),
        pltpu.VMEM((C,), jnp.int32),
        pltpu.VMEM_SHARED((NP, D_QK), jnp.float32),
        pltpu.VMEM_SHARED((NP // 8, D_QK), jnp.float32),
        pltpu.SemaphoreType.DMA,
    ],
)(_sc_body)


def _combine_body(wvp_ref, zp_ref, r_ref, out_ref):
    wv = wvp_ref[0] + wvp_ref[1]
    z16 = zp_ref[0] + zp_ref[1]
    z128 = jnp.dot(z16, r_ref[...], preferred_element_type=jnp.float32)
    out_ref[...] = wv / (z128 + 1e-8)


def _combine(wvp, zp, R):
    BM = 1000
    grid = (N_NODES // BM,)
    return pl.pallas_call(
        _combine_body,
        grid=grid,
        in_specs=[pl.BlockSpec((NC, BM, D_QK), lambda i: (0, i, 0)),
                  pl.BlockSpec((NC, BM, D_HEAD), lambda i: (0, i, 0)),
                  pl.BlockSpec((D_HEAD, D_QK), lambda i: (0, 0))],
        out_specs=pl.BlockSpec((BM, D_QK), lambda i: (i, 0)),
        out_shape=jax.ShapeDtypeStruct((N_NODES, D_QK), jnp.float32),
    )(wvp, zp, R)


def kernel(node_feats, edge_feats, edge_index, Wq, bq, Wk, bk, Wv, bv, We, be):
    src = edge_index[0].astype(jnp.int32)
    dst = edge_index[1].astype(jnp.int32)
    q, kv = _node_proj(node_feats, Wq, bq, Wk, bk, Wv, bv)
    pe = _edge_proj(edge_feats, We, be)
    e_out, wvp, zp = _sc_edge(kv, q, pe, src, dst)
    R = (jnp.arange(D_QK, dtype=jnp.int32)[None, :] // D_HEAD
         == jnp.arange(D_HEAD, dtype=jnp.int32)[:, None]).astype(jnp.float32)
    h_out = _combine(wvp.reshape(NC, NP, D_QK),
                     zp.reshape(NC, NP, D_HEAD), R)
    return (h_out.reshape(N_NODES, N_HEADS, D_HEAD),
            e_out.reshape(N_EDGES, N_HEADS, D_HEAD))


# speedup vs baseline: 15.0909x; 1.2744x over previous
import functools

import jax
import jax.numpy as jnp
from jax import lax
from jax.experimental import pallas as pl
from jax.experimental.pallas import tpu as pltpu
from jax.experimental.pallas import tpu_sc as plsc

N_NODES = 10000
N_EDGES = 320000
D_IN = 128
D_HEAD = 16
N_HEADS = 8
D_QK = D_HEAD * N_HEADS

NC = 2
NS = 16
NW = NC * NS
EPW = N_EDGES // NW
C = 16
NCHUNK = EPW // C
NP = 10240
RPT = NP // NS


def _proj_body(x_ref, wq_ref, bq_ref, wk_ref, bk_ref, wv_ref, bv_ref,
               q_ref, kv_ref):
    x = x_ref[...]
    q = jnp.dot(x, wq_ref[...], preferred_element_type=jnp.float32) + bq_ref[...]
    q_ref[...] = q * 0.25
    kv_ref[:, :D_QK] = (
        jnp.dot(x, wk_ref[...], preferred_element_type=jnp.float32) + bk_ref[...])
    kv_ref[:, D_QK:] = (
        jnp.dot(x, wv_ref[...], preferred_element_type=jnp.float32) + bv_ref[...])


def _node_proj(node_feats, Wq, bq, Wk, bk, Wv, bv):
    BM = 1000
    grid = (N_NODES // BM,)
    wspec = pl.BlockSpec((D_IN, D_QK), lambda i: (0, 0))
    bspec = pl.BlockSpec((1, D_QK), lambda i: (0, 0))
    return pl.pallas_call(
        _proj_body,
        grid=grid,
        in_specs=[pl.BlockSpec((BM, D_IN), lambda i: (i, 0)),
                  wspec, bspec, wspec, bspec, wspec, bspec],
        out_specs=[pl.BlockSpec((BM, D_QK), lambda i: (i, 0)),
                   pl.BlockSpec((BM, 2 * D_QK), lambda i: (i, 0))],
        out_shape=[jax.ShapeDtypeStruct((N_NODES, D_QK), jnp.float32),
                   jax.ShapeDtypeStruct((N_NODES, 2 * D_QK), jnp.float32)],
    )(node_feats, Wq, bq.reshape(1, -1), Wk, bk.reshape(1, -1),
      Wv, bv.reshape(1, -1))


def _edge_proj_body(x_ref, we_ref, be_ref, pe_ref):
    pe_ref[...] = (
        jnp.dot(x_ref[...], we_ref[...], preferred_element_type=jnp.float32)
        + be_ref[...])


def _edge_proj(edge_feats, We, be):
    BM = 2000
    grid = (N_EDGES // BM,)
    return pl.pallas_call(
        _edge_proj_body,
        grid=grid,
        in_specs=[pl.BlockSpec((BM, D_IN), lambda i: (i, 0)),
                  pl.BlockSpec((D_IN, D_QK), lambda i: (0, 0)),
                  pl.BlockSpec((1, D_QK), lambda i: (0, 0))],
        out_specs=pl.BlockSpec((BM, D_QK), lambda i: (i, 0)),
        out_shape=jax.ShapeDtypeStruct((N_EDGES, D_QK), jnp.float32),
    )(edge_feats, We, be.reshape(1, -1))


def _sc_body(kv_hbm, q_hbm, pe_hbm, src_hbm, dst_hbm,
             eout_hbm, wvp_hbm, zp_hbm,
             src0, src1, dst0, dst1, sd0, sd1, d80, d81,
             kv0, kv1, q0, q1, pe0, pe1, eo0, eo1, wv0, wv1, z0, z1,
             wv_acc, z_acc,
             si0, si1, sg0, sg1, so0, so1, sw0, sw1, sz0, sz1):
    cid = lax.axis_index("c")
    sid = lax.axis_index("s")
    wid = cid * NS + sid
    lane = lax.broadcasted_iota(jnp.int32, (D_HEAD,), 0)
    base0 = wid * EPW
    SRC = (src0, src1); DST = (dst0, dst1); SD = (sd0, sd1); D8 = (d80, d81)
    KV = (kv0, kv1); QB = (q0, q1); PE = (pe0, pe1)
    EO = (eo0, eo1); WV = (wv0, wv1); ZB = (z0, z1)
    SI = (si0, si1); SG = (sg0, sg1); SO = (so0, so1)
    SW = (sw0, sw1); SZ = (sz0, sz1)

    def zfill(i, _):
        for j in range(D_QK // D_HEAD):
            wv0[i, pl.ds(j * D_HEAD, D_HEAD)] = jnp.zeros((D_HEAD,),
                                                          jnp.float32)
            z0[i, pl.ds(j * D_HEAD, D_HEAD)] = jnp.zeros((D_HEAD,),
                                                         jnp.float32)
        return 0
    lax.fori_loop(0, C, zfill, 0)
    for b in range(RPT // C):
        pltpu.sync_copy(wv0, wv_acc.at[pl.ds(sid * RPT + b * C, C)])
    for b in range(RPT // (8 * C)):
        pltpu.sync_copy(z0, z_acc.at[pl.ds(sid * (RPT // 8) + b * C, C)])
    plsc.subcore_barrier()

    def idx_start(j, p):
        b = base0 + j * C
        pltpu.make_async_copy(src_hbm.at[pl.ds(b, C)], SRC[p], SI[p]).start()
        pltpu.make_async_copy(dst_hbm.at[pl.ds(b, C)], DST[p], SI[p]).start()

    def idx_wait(p):
        pltpu.make_async_copy(src_hbm.at[pl.ds(0, C)], SRC[p], SI[p]).wait()
        pltpu.make_async_copy(dst_hbm.at[pl.ds(0, C)], DST[p], SI[p]).wait()

    def gathers_start(j, p):
        b = base0 + j * C
        pltpu.make_async_copy(kv_hbm.at[SRC[p]], KV[p], SG[p]).start()
        pltpu.make_async_copy(q_hbm.at[DST[p]], QB[p], SG[p]).start()
        pltpu.make_async_copy(pe_hbm.at[pl.ds(b, C)], PE[p], SG[p]).start()

    def gathers_wait(p):
        pltpu.make_async_copy(kv_hbm.at[pl.ds(0, C)], KV[p], SG[p]).wait()
        pltpu.make_async_copy(q_hbm.at[pl.ds(0, C)], QB[p], SG[p]).wait()
        pltpu.make_async_copy(pe_hbm.at[pl.ds(0, C)], PE[p], SG[p]).wait()

    def outs_start(j, p):
        b = base0 + j * C
        pltpu.make_async_copy(EO[p], eout_hbm.at[pl.ds(b, C)], SO[p]).start()
        pltpu.make_async_copy(WV[p], wv_acc.at[SD[p]], SW[p]).start(add=True)
        pltpu.make_async_copy(ZB[p], z_acc.at[D8[p]], SZ[p]).start(add=True)

    def outs_wait(p):
        pltpu.make_async_copy(EO[p], eout_hbm.at[pl.ds(0, C)], SO[p]).wait()
        pltpu.make_async_copy(WV[p], wv_acc.at[SD[p]], SW[p]).wait()
        pltpu.make_async_copy(ZB[p], z_acc.at[D8[p]], SZ[p]).wait()

    def compute(p):
        kvb, qb, peb, eob, wvb, zb, sdb = (KV[p], QB[p], PE[p], EO[p],
                                           WV[p], ZB[p], SD[p])

        def edge(e, _):
            svals = jnp.zeros((D_HEAD,), jnp.float32)
            for h in range(N_HEADS):
                sl = pl.ds(h * D_HEAD, D_HEAD)
                k = kvb[e, sl]
                v = kvb[e, pl.ds(D_QK + h * D_HEAD, D_HEAD)]
                qv = qb[e, sl]
                pev = peb[e, sl]
                sc = jnp.clip(k * qv, -5.0, 5.0) * pev
                eob[e, sl] = sc
                tot = sc
                for sh in (8, 4, 2, 1):
                    tot = tot + tot[lane ^ sh]
                svec = jnp.exp(jnp.clip(tot, -5.0, 5.0))
                wvb[e, sl] = v * svec
                svals = jnp.where(lane == h, svec, svals)
            w = sdb[pl.ds(0, D_HEAD)]
            gvecf = (w[jnp.full((D_HEAD,), e, jnp.int32)] & 7
                     ).astype(jnp.float32)
            for g in range(8):
                ind = jnp.maximum(1.0 - jnp.abs(gvecf - float(g)), 0.0)
                zb[e, pl.ds(g * D_HEAD, D_HEAD)] = svals * ind
            return 0
        lax.fori_loop(0, C, edge, 0)

    def prep_scatter_idx(p):
        SD[p][...] = DST[p][...]
        D8[p][...] = lax.shift_right_logical(DST[p][...], 3)

    def step(j, p, first, do_np1, do_np2):
        if do_np1:
            idx_wait(p ^ 1)
            gathers_start(j + 1, p ^ 1)
        gathers_wait(p)
        if not first:
            outs_wait(p)
        prep_scatter_idx(p)
        compute(p)
        outs_start(j, p)
        if do_np2:
            idx_start(j + 2, p)

    idx_start(0, 0)
    idx_wait(0)
    gathers_start(0, 0)
    idx_start(1, 1)
    step(0, 0, True, True, True)
    step(1, 1, True, True, True)

    def pair(i, _):
        j = 2 * i
        step(j, 0, False, True, True)
        step(j + 1, 1, False, True, True)
        return 0
    lax.fori_loop(1, (NCHUNK - 3) // 2, pair, 0)

    step(NCHUNK - 3, 0, False, True, True)
    step(NCHUNK - 2, 1, False, True, False)
    step(NCHUNK - 1, 0, False, False, False)
    outs_wait(1)
    outs_wait(0)

    plsc.subcore_barrier()
    PUB = 8 * C
    for b in range(RPT // PUB):
        off = sid * RPT + b * PUB
        pltpu.sync_copy(wv_acc.at[pl.ds(off, PUB)],
                        wvp_hbm.at[pl.ds(cid * NP + off, PUB)])
    off = sid * (RPT // 8)
    pltpu.sync_copy(z_acc.at[pl.ds(off, RPT // 8)],
                    zp_hbm.at[pl.ds(cid * (NP // 8) + off, RPT // 8)])


_sc_edge = functools.partial(
    pl.kernel,
    out_type=[jax.ShapeDtypeStruct((N_EDGES, D_QK), jnp.float32),
              jax.ShapeDtypeStruct((NC * NP, D_QK), jnp.float32),
              jax.ShapeDtypeStruct((NC * (NP // 8), D_QK), jnp.float32)],
    mesh=plsc.VectorSubcoreMesh(core_axis_name="c", subcore_axis_name="s",
                                num_cores=NC, num_subcores=NS),
    scratch_types=(
        [pltpu.VMEM((C,), jnp.int32) for _ in range(8)]
        + [pltpu.VMEM((C, 2 * D_QK), jnp.float32) for _ in range(2)]
        + [pltpu.VMEM((C, D_QK), jnp.float32) for _ in range(10)]
        + [pltpu.VMEM_SHARED((NP, D_QK), jnp.float32),
           pltpu.VMEM_SHARED((NP // 8, D_QK), jnp.float32)]
        + [pltpu.SemaphoreType.DMA for _ in range(10)]
    ),
)(_sc_body)


def _combine_body(wvp_ref, zp_ref, r_ref, out_ref):
    wv = wvp_ref[0] + wvp_ref[1]
    z16 = zp_ref[0] + zp_ref[1]
    z128 = jnp.dot(z16, r_ref[...], preferred_element_type=jnp.float32)
    out_ref[...] = wv / (z128 + 1e-8)


def _combine(wvp, zp, R):
    BM = 1000
    grid = (N_NODES // BM,)
    return pl.pallas_call(
        _combine_body,
        grid=grid,
        in_specs=[pl.BlockSpec((NC, BM, D_QK), lambda i: (0, i, 0)),
                  pl.BlockSpec((NC, BM, D_HEAD), lambda i: (0, i, 0)),
                  pl.BlockSpec((D_HEAD, D_QK), lambda i: (0, 0))],
        out_specs=pl.BlockSpec((BM, D_QK), lambda i: (i, 0)),
        out_shape=jax.ShapeDtypeStruct((N_NODES, D_QK), jnp.float32),
    )(wvp, zp, R)


def kernel(node_feats, edge_feats, edge_index, Wq, bq, Wk, bk, Wv, bv, We, be):
    src = edge_index[0].astype(jnp.int32)
    dst = edge_index[1].astype(jnp.int32)
    q, kv = _node_proj(node_feats, Wq, bq, Wk, bk, Wv, bv)
    pe = _edge_proj(edge_feats, We, be)
    e_out, wvp, zp = _sc_edge(kv, q, pe, src, dst)
    R = (jnp.arange(D_QK, dtype=jnp.int32)[None, :] // D_HEAD
         == jnp.arange(D_HEAD, dtype=jnp.int32)[:, None]).astype(jnp.float32)
    h_out = _combine(wvp.reshape(NC, NP, D_QK),
                     zp.reshape(NC, NP, D_HEAD), R)
    return (h_out.reshape(N_NODES, N_HEADS, D_HEAD),
            e_out.reshape(N_EDGES, N_HEADS, D_HEAD))


# speedup vs baseline: 47.3407x; 3.1370x over previous
import functools

import jax
import jax.numpy as jnp
from jax import lax
from jax.experimental import pallas as pl
from jax.experimental.pallas import tpu as pltpu
from jax.experimental.pallas import tpu_sc as plsc

N_NODES = 10000
N_EDGES = 320000
D_IN = 128
D_HEAD = 16
N_HEADS = 8
D_QK = D_HEAD * N_HEADS

NC = 2
NS = 16
NW = NC * NS
EPW = N_EDGES // NW
C = 16
NCHUNK = EPW // C
NP = 10240
RPT = NP // NS


def _proj_body(x_ref, wq_ref, bq_ref, wk_ref, bk_ref, wv_ref, bv_ref,
               q_ref, kv_ref):
    x = x_ref[...]
    q = jnp.dot(x, wq_ref[...], preferred_element_type=jnp.float32) + bq_ref[...]
    q_ref[...] = q * 0.25
    kv_ref[:, :D_QK] = (
        jnp.dot(x, wk_ref[...], preferred_element_type=jnp.float32) + bk_ref[...])
    kv_ref[:, D_QK:] = (
        jnp.dot(x, wv_ref[...], preferred_element_type=jnp.float32) + bv_ref[...])


def _node_proj(node_feats, Wq, bq, Wk, bk, Wv, bv):
    BM = 1000
    grid = (N_NODES // BM,)
    wspec = pl.BlockSpec((D_IN, D_QK), lambda i: (0, 0))
    bspec = pl.BlockSpec((1, D_QK), lambda i: (0, 0))
    return pl.pallas_call(
        _proj_body,
        grid=grid,
        in_specs=[pl.BlockSpec((BM, D_IN), lambda i: (i, 0)),
                  wspec, bspec, wspec, bspec, wspec, bspec],
        out_specs=[pl.BlockSpec((BM, D_QK), lambda i: (i, 0)),
                   pl.BlockSpec((BM, 2 * D_QK), lambda i: (i, 0))],
        out_shape=[jax.ShapeDtypeStruct((N_NODES, D_QK), jnp.float32),
                   jax.ShapeDtypeStruct((N_NODES, 2 * D_QK), jnp.float32)],
    )(node_feats, Wq, bq.reshape(1, -1), Wk, bk.reshape(1, -1),
      Wv, bv.reshape(1, -1))


def _edge_proj_body(x_ref, we_ref, be_ref, pe_ref):
    pe_ref[...] = (
        jnp.dot(x_ref[...], we_ref[...], preferred_element_type=jnp.float32)
        + be_ref[...])


def _edge_proj(edge_feats, We, be):
    BM = 2000
    grid = (N_EDGES // BM,)
    return pl.pallas_call(
        _edge_proj_body,
        grid=grid,
        in_specs=[pl.BlockSpec((BM, D_IN), lambda i: (i, 0)),
                  pl.BlockSpec((D_IN, D_QK), lambda i: (0, 0)),
                  pl.BlockSpec((1, D_QK), lambda i: (0, 0))],
        out_specs=pl.BlockSpec((BM, D_QK), lambda i: (i, 0)),
        out_shape=jax.ShapeDtypeStruct((N_EDGES, D_QK), jnp.float32),
    )(edge_feats, We, be.reshape(1, -1))


def _sc_body(kv_hbm, q_hbm, pe_hbm, src_hbm, dst_hbm,
             eout_hbm, wvp_hbm, zp_hbm,
             src0, src1, dst0, dst1, sd0, sd1, d80, d81,
             kv0, kv1, q0, q1, pe0, pe1, eo0, eo1, wv0, wv1, z0, z1,
             wv_acc, z_acc,
             si0, si1, sg0, sg1, so0, so1, sw0, sw1, sz0, sz1):
    cid = lax.axis_index("c")
    sid = lax.axis_index("s")
    wid = cid * NS + sid
    lane = lax.broadcasted_iota(jnp.int32, (D_HEAD,), 0)
    base0 = wid * EPW
    SRC = (src0, src1); DST = (dst0, dst1); SD = (sd0, sd1); D8 = (d80, d81)
    KV = (kv0, kv1); QB = (q0, q1); PE = (pe0, pe1)
    EO = (eo0, eo1); WV = (wv0, wv1); ZB = (z0, z1)
    SI = (si0, si1); SG = (sg0, sg1); SO = (so0, so1)
    SW = (sw0, sw1); SZ = (sz0, sz1)

    def zfill(i, _):
        for j in range(D_QK // D_HEAD):
            wv0[i, pl.ds(j * D_HEAD, D_HEAD)] = jnp.zeros((D_HEAD,),
                                                          jnp.float32)
            z0[i, pl.ds(j * D_HEAD, D_HEAD)] = jnp.zeros((D_HEAD,),
                                                         jnp.float32)
        return 0
    lax.fori_loop(0, C, zfill, 0)
    for b in range(RPT // C):
        pltpu.sync_copy(wv0, wv_acc.at[pl.ds(sid * RPT + b * C, C)])
    for b in range(RPT // (8 * C)):
        pltpu.sync_copy(z0, z_acc.at[pl.ds(sid * (RPT // 8) + b * C, C)])
    plsc.subcore_barrier()

    def idx_start(j, p):
        b = base0 + j * C
        pltpu.make_async_copy(src_hbm.at[pl.ds(b, C)], SRC[p], SI[p]).start()
        pltpu.make_async_copy(dst_hbm.at[pl.ds(b, C)], DST[p], SI[p]).start()

    def idx_wait(p):
        pltpu.make_async_copy(src_hbm.at[pl.ds(0, C)], SRC[p], SI[p]).wait()
        pltpu.make_async_copy(dst_hbm.at[pl.ds(0, C)], DST[p], SI[p]).wait()

    def gathers_start(j, p):
        b = base0 + j * C
        pltpu.make_async_copy(kv_hbm.at[SRC[p]], KV[p], SG[p]).start()
        pltpu.make_async_copy(q_hbm.at[DST[p]], QB[p], SG[p]).start()
        pltpu.make_async_copy(pe_hbm.at[pl.ds(b, C)], PE[p], SG[p]).start()

    def gathers_wait(p):
        pltpu.make_async_copy(kv_hbm.at[pl.ds(0, C)], KV[p], SG[p]).wait()
        pltpu.make_async_copy(q_hbm.at[pl.ds(0, C)], QB[p], SG[p]).wait()
        pltpu.make_async_copy(pe_hbm.at[pl.ds(0, C)], PE[p], SG[p]).wait()

    def outs_start(j, p):
        b = base0 + j * C
        pltpu.make_async_copy(EO[p], eout_hbm.at[pl.ds(b, C)], SO[p]).start()
        pltpu.make_async_copy(WV[p], wv_acc.at[SD[p]], SW[p]).start(add=True)
        pltpu.make_async_copy(ZB[p], z_acc.at[D8[p]], SZ[p]).start(add=True)

    def outs_wait(p):
        pltpu.make_async_copy(EO[p], eout_hbm.at[pl.ds(0, C)], SO[p]).wait()
        pltpu.make_async_copy(WV[p], wv_acc.at[SD[p]], SW[p]).wait()
        pltpu.make_async_copy(ZB[p], z_acc.at[D8[p]], SZ[p]).wait()

    def compute(p):
        kvb, qb, peb, eob, wvb, zb, sdb = (KV[p], QB[p], PE[p], EO[p],
                                           WV[p], ZB[p], SD[p])

        @functools.partial(plsc.parallel_loop, 0, C, unroll=2)
        def edge(e):
            svals = jnp.zeros((D_HEAD,), jnp.float32)
            for h in range(N_HEADS):
                sl = pl.ds(h * D_HEAD, D_HEAD)
                k = kvb[e, sl]
                v = kvb[e, pl.ds(D_QK + h * D_HEAD, D_HEAD)]
                qv = qb[e, sl]
                pev = peb[e, sl]
                sc = jnp.clip(k * qv, -5.0, 5.0) * pev
                eob[e, sl] = sc
                tot = sc
                for sh in (8, 4, 2, 1):
                    tot = tot + tot[lane ^ sh]
                svec = jnp.exp(jnp.clip(tot, -5.0, 5.0))
                wvb[e, sl] = v * svec
                svals = jnp.where(lane == h, svec, svals)
            w = sdb[pl.ds(0, D_HEAD)]
            gvecf = (w[jnp.full((D_HEAD,), e, jnp.int32)] & 7
                     ).astype(jnp.float32)
            for g in range(8):
                ind = jnp.maximum(1.0 - jnp.abs(gvecf - float(g)), 0.0)
                zb[e, pl.ds(g * D_HEAD, D_HEAD)] = svals * ind

    def prep_scatter_idx(p):
        SD[p][...] = DST[p][...]
        D8[p][...] = lax.shift_right_logical(DST[p][...], 3)

    def step(j, p, first, do_np1, do_np2):
        if do_np1:
            idx_wait(p ^ 1)
            gathers_start(j + 1, p ^ 1)
        gathers_wait(p)
        if not first:
            outs_wait(p)
        prep_scatter_idx(p)
        compute(p)
        outs_start(j, p)
        if do_np2:
            idx_start(j + 2, p)

    idx_start(0, 0)
    idx_wait(0)
    gathers_start(0, 0)
    idx_start(1, 1)
    step(0, 0, True, True, True)
    step(1, 1, True, True, True)

    def pair(i, _):
        j = 2 * i
        step(j, 0, False, True, True)
        step(j + 1, 1, False, True, True)
        return 0
    lax.fori_loop(1, (NCHUNK - 3) // 2, pair, 0)

    step(NCHUNK - 3, 0, False, True, True)
    step(NCHUNK - 2, 1, False, True, False)
    step(NCHUNK - 1, 0, False, False, False)
    outs_wait(1)
    outs_wait(0)

    plsc.subcore_barrier()
    PUB = 8 * C
    for b in range(RPT // PUB):
        off = sid * RPT + b * PUB
        pltpu.sync_copy(wv_acc.at[pl.ds(off, PUB)],
                        wvp_hbm.at[pl.ds(cid * NP + off, PUB)])
    off = sid * (RPT // 8)
    pltpu.sync_copy(z_acc.at[pl.ds(off, RPT // 8)],
                    zp_hbm.at[pl.ds(cid * (NP // 8) + off, RPT // 8)])


_sc_edge = functools.partial(
    pl.kernel,
    out_type=[jax.ShapeDtypeStruct((N_EDGES, D_QK), jnp.float32),
              jax.ShapeDtypeStruct((NC * NP, D_QK), jnp.float32),
              jax.ShapeDtypeStruct((NC * (NP // 8), D_QK), jnp.float32)],
    mesh=plsc.VectorSubcoreMesh(core_axis_name="c", subcore_axis_name="s",
                                num_cores=NC, num_subcores=NS),
    scratch_types=(
        [pltpu.VMEM((C,), jnp.int32) for _ in range(8)]
        + [pltpu.VMEM((C, 2 * D_QK), jnp.float32) for _ in range(2)]
        + [pltpu.VMEM((C, D_QK), jnp.float32) for _ in range(10)]
        + [pltpu.VMEM_SHARED((NP, D_QK), jnp.float32),
           pltpu.VMEM_SHARED((NP // 8, D_QK), jnp.float32)]
        + [pltpu.SemaphoreType.DMA for _ in range(10)]
    ),
)(_sc_body)


def _combine_body(wvp_ref, zp_ref, r_ref, out_ref):
    wv = wvp_ref[0] + wvp_ref[1]
    z16 = zp_ref[0] + zp_ref[1]
    z128 = jnp.dot(z16, r_ref[...], preferred_element_type=jnp.float32)
    out_ref[...] = wv / (z128 + 1e-8)


def _combine(wvp, zp, R):
    BM = 1000
    grid = (N_NODES // BM,)
    return pl.pallas_call(
        _combine_body,
        grid=grid,
        in_specs=[pl.BlockSpec((NC, BM, D_QK), lambda i: (0, i, 0)),
                  pl.BlockSpec((NC, BM, D_HEAD), lambda i: (0, i, 0)),
                  pl.BlockSpec((D_HEAD, D_QK), lambda i: (0, 0))],
        out_specs=pl.BlockSpec((BM, D_QK), lambda i: (i, 0)),
        out_shape=jax.ShapeDtypeStruct((N_NODES, D_QK), jnp.float32),
    )(wvp, zp, R)


def kernel(node_feats, edge_feats, edge_index, Wq, bq, Wk, bk, Wv, bv, We, be):
    src = edge_index[0].astype(jnp.int32)
    dst = edge_index[1].astype(jnp.int32)
    q, kv = _node_proj(node_feats, Wq, bq, Wk, bk, Wv, bv)
    pe = _edge_proj(edge_feats, We, be)
    e_out, wvp, zp = _sc_edge(kv, q, pe, src, dst)
    R = (jnp.arange(D_QK, dtype=jnp.int32)[None, :] // D_HEAD
         == jnp.arange(D_HEAD, dtype=jnp.int32)[:, None]).astype(jnp.float32)
    h_out = _combine(wvp.reshape(NC, NP, D_QK),
                     zp.reshape(NC, NP, D_HEAD), R)
    return (h_out.reshape(N_NODES, N_HEADS, D_HEAD),
            e_out.reshape(N_EDGES, N_HEADS, D_HEAD))
